# baseline (speedup 1.0000x reference)
import functools

import jax
import jax.numpy as jnp
from jax import lax
from jax.experimental import pallas as pl
from jax.experimental.pallas import tpu as pltpu
from jax.experimental.pallas import tpu_sc as plsc

B, V, N = 32, 4, 8192
H = W = 800

BB = 8
GRID = B // BB
ROWS = BB * V


def _proj_body(e_ref, kbd_ref, pc_ref, idx_ref):
    ones = jnp.ones((1, N), jnp.float32)
    parts = []
    for bp in range(BB):
        parts.append(pc_ref[bp])
        parts.append(ones)
    hom = jnp.concatenate(parts, axis=0)
    e = e_ref[0]
    pt = lax.dot_general(
        e, hom, (((1,), (0,)), ((), ())),
        preferred_element_type=jnp.float32,
    )
    p2 = lax.dot_general(
        kbd_ref[...], pt, (((1,), (0,)), ((), ())),
        preferred_element_type=jnp.float32,
    )
    px = p2[0:ROWS]
    py = p2[ROWS:2 * ROWS]
    pz = p2[2 * ROWS:3 * ROWS]
    xs = px / pz
    ys = py / pz
    xi = jnp.clip(jnp.round(xs).astype(jnp.int32), 0, W - 1)
    yi = jnp.clip(jnp.round(ys).astype(jnp.int32), 0, H - 1)
    lrow = lax.broadcasted_iota(jnp.int32, (ROWS, N), 0)
    idx_ref[...] = (lrow * H + yi) * W + xi


def _project(point_cloud, K, ext_trans):
    ext_r = ext_trans.reshape(GRID, BB, V, 3, 4)
    eye = jnp.eye(BB, dtype=ext_trans.dtype)
    e_all = jnp.einsum("gbvij,bc->gibvcj", ext_r, eye)
    e_all = e_all.reshape(GRID, 3 * ROWS, 4 * BB)
    kbd = jnp.kron(K, jnp.eye(ROWS, dtype=K.dtype))
    return pl.pallas_call(
        _proj_body,
        grid=(GRID,),
        in_specs=[
            pl.BlockSpec((1, 3 * ROWS, 4 * BB), lambda g: (g, 0, 0)),
            pl.BlockSpec((3 * ROWS, 3 * ROWS), lambda g: (0, 0)),
            pl.BlockSpec((BB, 3, N), lambda g: (g, 0, 0)),
        ],
        out_specs=pl.BlockSpec((ROWS, N), lambda g: (g, 0)),
        out_shape=jax.ShapeDtypeStruct((B * V, N), jnp.int32),
    )(e_all, kbd, point_cloud)


NC = 2
NW = 32
CHUNK = 1024
NCHUNK = N // CHUNK
FLAT_C = BB * V * H * W


def _make_sc_body(k):
    def _sc_gather_body(img_hbm, idx_hbm, out_hbm, idx_v, vals_v, sem):
        wid = lax.axis_index("s") * NC + lax.axis_index("c")
        row = k * ROWS + wid
        pltpu.sync_copy(idx_hbm.at[row], idx_v)

        def fire(c, carry):
            o = c * CHUNK
            pltpu.async_copy(
                img_hbm.at[idx_v.at[pl.ds(o, CHUNK)]],
                vals_v.at[pl.ds(o, CHUNK)],
                sem,
            )
            return carry

        lax.fori_loop(0, NCHUNK, fire, 0)
        pltpu.make_async_copy(out_hbm.at[wid], vals_v, sem).wait()
        pltpu.sync_copy(vals_v, out_hbm.at[wid])

    return _sc_gather_body


@functools.cache
def _sc_gather(k):
    mesh = plsc.VectorSubcoreMesh(core_axis_name="c", subcore_axis_name="s")
    return pl.kernel(
        _make_sc_body(k),
        out_type=jax.ShapeDtypeStruct((ROWS, N), jnp.float32),
        mesh=mesh,
        scratch_types=[
            pltpu.VMEM((N,), jnp.int32),
            pltpu.VMEM((N,), jnp.float32),
            pltpu.SemaphoreType.DMA,
        ],
    )


def kernel(point_cloud, img_seg, K, ext_trans):
    idx = _project(point_cloud, K, ext_trans)
    outs = []
    for k in range(GRID):
        flat_k = img_seg[k * BB:(k + 1) * BB].reshape(FLAT_C)
        outs.append(_sc_gather(k)(flat_k, idx))
    vals = jnp.concatenate(outs, axis=0)
    return vals.reshape(B, V, N).astype(bool)


# speedup vs baseline: 1.4156x; 1.4156x over previous
import functools

import jax
import jax.numpy as jnp
from jax import lax
from jax.experimental import pallas as pl
from jax.experimental.pallas import tpu as pltpu
from jax.experimental.pallas import tpu_sc as plsc

B, V, N = 32, 4, 8192
H = W = 800

BB = 8
GRID = B // BB
ROWS = BB * V

XT = 7
YT = XT * 8 * 128
SLAB = (H // 8) * YT
FLAT = B * V * H * W


def _proj_body(e_ref, kbd_ref, pc_ref, idx_ref):
    g = pl.program_id(0)
    ones = jnp.ones((1, N), jnp.float32)
    parts = []
    for bp in range(BB):
        parts.append(pc_ref[bp])
        parts.append(ones)
    hom = jnp.concatenate(parts, axis=0)
    e = e_ref[0]
    pt = lax.dot_general(
        e, hom, (((1,), (0,)), ((), ())),
        preferred_element_type=jnp.float32,
    )
    p2 = lax.dot_general(
        kbd_ref[...], pt, (((1,), (0,)), ((), ())),
        preferred_element_type=jnp.float32,
    )
    px = p2[0:ROWS]
    py = p2[ROWS:2 * ROWS]
    pz = p2[2 * ROWS:3 * ROWS]
    xs = px / pz
    ys = py / pz
    xi = jnp.clip(jnp.round(xs).astype(jnp.int32), 0, W - 1)
    yi = jnp.clip(jnp.round(ys).astype(jnp.int32), 0, H - 1)
    row = lax.broadcasted_iota(jnp.int32, (ROWS, N), 0) + g * ROWS
    idx_ref[...] = row * (H * W) + yi * W + xi


def _project(point_cloud, K, ext_trans):
    ext_r = ext_trans.reshape(GRID, BB, V, 3, 4)
    eye = jnp.eye(BB, dtype=ext_trans.dtype)
    e_all = jnp.einsum("gbvij,bc->gibvcj", ext_r, eye)
    e_all = e_all.reshape(GRID, 3 * ROWS, 4 * BB)
    kbd = jnp.kron(K, jnp.eye(ROWS, dtype=K.dtype))
    return pl.pallas_call(
        _proj_body,
        grid=(GRID,),
        in_specs=[
            pl.BlockSpec((1, 3 * ROWS, 4 * BB), lambda g: (g, 0, 0)),
            pl.BlockSpec((3 * ROWS, 3 * ROWS), lambda g: (0, 0)),
            pl.BlockSpec((BB, 3, N), lambda g: (g, 0, 0)),
        ],
        out_specs=pl.BlockSpec((ROWS, N), lambda g: (g, 0)),
        out_shape=jax.ShapeDtypeStruct((B * V, N), jnp.int32),
    )(e_all, kbd, point_cloud)


NC = 2
NW = 32
RPW = (B * V) // NW
PPW = RPW * N


def _sc_gather_body(img_hbm, idx_hbm, out_hbm, idx_v, vals_v, sem):
    wid = lax.axis_index("s") * NC + lax.axis_index("c")
    base = wid * RPW
    for v in range(RPW):
        pltpu.sync_copy(idx_hbm.at[base + v], idx_v.at[pl.ds(v * N, N)])
    pltpu.async_copy(img_hbm.at[idx_v], vals_v, sem).wait()
    for v in range(RPW):
        pltpu.sync_copy(vals_v.at[pl.ds(v * N, N)], out_hbm.at[base + v])


@functools.cache
def _sc_gather():
    mesh = plsc.VectorSubcoreMesh(core_axis_name="c", subcore_axis_name="s")
    return pl.kernel(
        _sc_gather_body,
        out_type=jax.ShapeDtypeStruct((B * V, N), jnp.float32),
        mesh=mesh,
        scratch_types=[
            pltpu.VMEM((RPW * N,), jnp.int32),
            pltpu.VMEM((RPW * N,), jnp.float32),
            pltpu.SemaphoreType.DMA,
        ],
    )


def kernel(point_cloud, img_seg, K, ext_trans):
    idx = _project(point_cloud, K, ext_trans)
    vals = _sc_gather()(img_seg.reshape(B * V * H * W), idx)
    return vals.reshape(B, V, N).astype(bool)


# speedup vs baseline: 3.6490x; 2.5777x over previous
import functools

import jax
import jax.numpy as jnp
from jax import lax
from jax.experimental import pallas as pl
from jax.experimental.pallas import tpu as pltpu
from jax.experimental.pallas import tpu_sc as plsc

B, V, N = 32, 4, 8192
H = W = 800
WPR = W // 32
BROWS = V * WPR


def _pack_body(wlo_ref, whi_ref, img_ref, bits_ref):
    for v in range(V):
        m = img_ref[0, v]
        lo = lax.dot_general(
            wlo_ref[...], m, (((0,), (1,)), ((), ())),
            preferred_element_type=jnp.float32,
        )
        hi = lax.dot_general(
            whi_ref[...], m, (((0,), (1,)), ((), ())),
            preferred_element_type=jnp.float32,
        )
        words = lo.astype(jnp.int32) | (hi.astype(jnp.int32) << 16)
        bits_ref[0, v * WPR:(v + 1) * WPR, :] = words


def _pack(img_seg):
    j = jnp.arange(W)
    c = j >> 5
    s = j & 31
    onehot = (c[:, None] == jnp.arange(WPR)[None, :]).astype(jnp.float32)
    wlo = onehot * jnp.where(s < 16, 2.0 ** s, 0.0)[:, None]
    whi = onehot * jnp.where(s >= 16, 2.0 ** (s - 16), 0.0)[:, None]
    return pl.pallas_call(
        _pack_body,
        grid=(B,),
        in_specs=[
            pl.BlockSpec((W, WPR), lambda b: (0, 0)),
            pl.BlockSpec((W, WPR), lambda b: (0, 0)),
            pl.BlockSpec((1, V, H, W), lambda b: (b, 0, 0, 0)),
        ],
        out_specs=pl.BlockSpec((1, BROWS, H), lambda b: (b, 0, 0)),
        out_shape=jax.ShapeDtypeStruct((B, BROWS, H), jnp.int32),
    )(wlo.astype(jnp.float32), whi.astype(jnp.float32), img_seg)


BB = 8
GRID = B // BB
ROWS = BB * V


def _proj_body(e_ref, kbd_ref, pc_ref, idx_ref):
    ones = jnp.ones((1, N), jnp.float32)
    parts = []
    for bp in range(BB):
        parts.append(pc_ref[bp])
        parts.append(ones)
    hom = jnp.concatenate(parts, axis=0)
    e = e_ref[0]
    pt = lax.dot_general(
        e, hom, (((1,), (0,)), ((), ())),
        preferred_element_type=jnp.float32,
    )
    p2 = lax.dot_general(
        kbd_ref[...], pt, (((1,), (0,)), ((), ())),
        preferred_element_type=jnp.float32,
    )
    px = p2[0:ROWS]
    py = p2[ROWS:2 * ROWS]
    pz = p2[2 * ROWS:3 * ROWS]
    xs = px / pz
    ys = py / pz
    xi = jnp.clip(jnp.round(xs).astype(jnp.int32), 0, W - 1)
    yi = jnp.clip(jnp.round(ys).astype(jnp.int32), 0, H - 1)
    vv = lax.broadcasted_iota(jnp.int32, (ROWS, N), 0) & 3
    brow = vv * WPR + (xi >> 5)
    idx_ref[...] = (brow << 15) | (yi << 5) | (xi & 31)


def _project(point_cloud, K, ext_trans):
    ext_r = ext_trans.reshape(GRID, BB, V, 3, 4)
    eye = jnp.eye(BB, dtype=ext_trans.dtype)
    e_all = jnp.einsum("gbvij,bc->gibvcj", ext_r, eye)
    e_all = e_all.reshape(GRID, 3 * ROWS, 4 * BB)
    kbd = jnp.kron(K, jnp.eye(ROWS, dtype=K.dtype))
    return pl.pallas_call(
        _proj_body,
        grid=(GRID,),
        in_specs=[
            pl.BlockSpec((1, 3 * ROWS, 4 * BB), lambda g: (g, 0, 0)),
            pl.BlockSpec((3 * ROWS, 3 * ROWS), lambda g: (0, 0)),
            pl.BlockSpec((BB, 3, N), lambda g: (g, 0, 0)),
        ],
        out_specs=pl.BlockSpec((ROWS, N), lambda g: (g, 0)),
        out_shape=jax.ShapeDtypeStruct((B * V, N), jnp.int32),
    )(e_all, kbd, point_cloud)


NC = 2
NW = 32


def _sc_gather_body(bits_hbm, idx_hbm, out_hbm, bits_v, pv, ov, sem):
    wid = lax.axis_index("s") * NC + lax.axis_index("c")
    pltpu.sync_copy(bits_hbm.at[wid], bits_v)

    def per_v(v, carry):
        row = wid * V + v
        pltpu.sync_copy(idx_hbm.at[row], pv)

        def body(i, c):
            o = i * 16
            p = pv[pl.ds(o, 16)]
            w = plsc.load_gather(bits_v, [p >> 15, (p >> 5) & 1023])
            bit = (w >> (p & 31)) & 1
            ov[pl.ds(o, 16)] = bit.astype(jnp.float32)
            return c

        lax.fori_loop(0, N // 16, body, 0)
        pltpu.sync_copy(ov, out_hbm.at[row])
        return carry

    lax.fori_loop(0, V, per_v, 0)


@functools.cache
def _sc_gather():
    mesh = plsc.VectorSubcoreMesh(core_axis_name="c", subcore_axis_name="s")
    return pl.kernel(
        _sc_gather_body,
        out_type=jax.ShapeDtypeStruct((B * V, N), jnp.float32),
        mesh=mesh,
        compiler_params=pltpu.CompilerParams(needs_layout_passes=False),
        scratch_types=[
            pltpu.VMEM((BROWS, H), jnp.int32),
            pltpu.VMEM((N,), jnp.int32),
            pltpu.VMEM((N,), jnp.float32),
            pltpu.SemaphoreType.DMA,
        ],
    )


def kernel(point_cloud, img_seg, K, ext_trans):
    bits = _pack(img_seg)
    idx = _project(point_cloud, K, ext_trans)
    vals = _sc_gather()(bits, idx)
    return vals.reshape(B, V, N).astype(bool)


# speedup vs baseline: 4.0093x; 1.0988x over previous
import functools

import jax
import jax.numpy as jnp
from jax import lax
from jax.experimental import pallas as pl
from jax.experimental.pallas import tpu as pltpu
from jax.experimental.pallas import tpu_sc as plsc

B, V, N = 32, 4, 8192
H = W = 800
WPR = W // 32
BROWS = V * WPR


def _pack_body(wlo_ref, whi_ref, img_ref, bits_ref):
    for v in range(V):
        m = img_ref[0, v]
        lo = lax.dot_general(
            wlo_ref[...], m, (((1,), (0,)), ((), ())),
            preferred_element_type=jnp.float32,
        )
        hi = lax.dot_general(
            whi_ref[...], m, (((1,), (0,)), ((), ())),
            preferred_element_type=jnp.float32,
        )
        words = lo.astype(jnp.int32) | (hi.astype(jnp.int32) << 16)
        bits_ref[0, v * WPR:(v + 1) * WPR, :] = words


def _pack(img_seg):
    j = jnp.arange(H)
    c = j >> 5
    s = j & 31
    onehot = (jnp.arange(WPR)[:, None] == c[None, :]).astype(jnp.float32)
    wlo = onehot * jnp.where(s < 16, 2.0 ** s, 0.0)[None, :]
    whi = onehot * jnp.where(s >= 16, 2.0 ** (s - 16), 0.0)[None, :]
    return pl.pallas_call(
        _pack_body,
        grid=(B,),
        in_specs=[
            pl.BlockSpec((WPR, H), lambda b: (0, 0)),
            pl.BlockSpec((WPR, H), lambda b: (0, 0)),
            pl.BlockSpec((1, V, H, W), lambda b: (b, 0, 0, 0)),
        ],
        out_specs=pl.BlockSpec((1, BROWS, H), lambda b: (b, 0, 0)),
        out_shape=jax.ShapeDtypeStruct((B, BROWS, H), jnp.int32),
    )(wlo.astype(jnp.float32), whi.astype(jnp.float32), img_seg)


BB = 8
GRID = B // BB
ROWS = BB * V


def _proj_body(e_ref, kbd_ref, pc_ref, idx_ref):
    ones = jnp.ones((1, N), jnp.float32)
    parts = []
    for bp in range(BB):
        parts.append(pc_ref[bp])
        parts.append(ones)
    hom = jnp.concatenate(parts, axis=0)
    e = e_ref[0]
    pt = lax.dot_general(
        e, hom, (((1,), (0,)), ((), ())),
        preferred_element_type=jnp.float32,
    )
    p2 = lax.dot_general(
        kbd_ref[...], pt, (((1,), (0,)), ((), ())),
        preferred_element_type=jnp.float32,
    )
    px = p2[0:ROWS]
    py = p2[ROWS:2 * ROWS]
    pz = p2[2 * ROWS:3 * ROWS]
    xs = px / pz
    ys = py / pz
    xi = jnp.clip(jnp.round(xs).astype(jnp.int32), 0, W - 1)
    yi = jnp.clip(jnp.round(ys).astype(jnp.int32), 0, H - 1)
    vv = lax.broadcasted_iota(jnp.int32, (ROWS, N), 0) & 3
    brow = vv * WPR + (yi >> 5)
    idx_ref[...] = (brow << 15) | (xi << 5) | (yi & 31)


def _project(point_cloud, K, ext_trans):
    ext_r = ext_trans.reshape(GRID, BB, V, 3, 4)
    eye = jnp.eye(BB, dtype=ext_trans.dtype)
    e_all = jnp.einsum("gbvij,bc->gibvcj", ext_r, eye)
    e_all = e_all.reshape(GRID, 3 * ROWS, 4 * BB)
    kbd = jnp.kron(K, jnp.eye(ROWS, dtype=K.dtype))
    return pl.pallas_call(
        _proj_body,
        grid=(GRID,),
        in_specs=[
            pl.BlockSpec((1, 3 * ROWS, 4 * BB), lambda g: (g, 0, 0)),
            pl.BlockSpec((3 * ROWS, 3 * ROWS), lambda g: (0, 0)),
            pl.BlockSpec((BB, 3, N), lambda g: (g, 0, 0)),
        ],
        out_specs=pl.BlockSpec((ROWS, N), lambda g: (g, 0)),
        out_shape=jax.ShapeDtypeStruct((B * V, N), jnp.int32),
    )(e_all, kbd, point_cloud)


NC = 2
NW = 32


def _sc_gather_body(bits_hbm, idx_hbm, out_hbm, bits_v, pv, ov, sem):
    wid = lax.axis_index("s") * NC + lax.axis_index("c")
    pltpu.sync_copy(bits_hbm.at[wid], bits_v)

    def per_v(v, carry):
        row = wid * V + v
        pltpu.sync_copy(idx_hbm.at[row], pv)

        def body(i, c):
            o = i * 16
            p = pv[pl.ds(o, 16)]
            w = plsc.load_gather(bits_v, [p >> 15, (p >> 5) & 1023])
            bit = (w >> (p & 31)) & 1
            ov[pl.ds(o, 16)] = bit.astype(jnp.float32)
            return c

        lax.fori_loop(0, N // 16, body, 0)
        pltpu.sync_copy(ov, out_hbm.at[row])
        return carry

    lax.fori_loop(0, V, per_v, 0)


@functools.cache
def _sc_gather():
    mesh = plsc.VectorSubcoreMesh(core_axis_name="c", subcore_axis_name="s")
    return pl.kernel(
        _sc_gather_body,
        out_type=jax.ShapeDtypeStruct((B * V, N), jnp.float32),
        mesh=mesh,
        compiler_params=pltpu.CompilerParams(needs_layout_passes=False),
        scratch_types=[
            pltpu.VMEM((BROWS, H), jnp.int32),
            pltpu.VMEM((N,), jnp.int32),
            pltpu.VMEM((N,), jnp.float32),
            pltpu.SemaphoreType.DMA,
        ],
    )


def kernel(point_cloud, img_seg, K, ext_trans):
    bits = _pack(img_seg)
    idx = _project(point_cloud, K, ext_trans)
    vals = _sc_gather()(bits, idx)
    return vals.reshape(B, V, N).astype(bool)


# speedup vs baseline: 4.0141x; 1.0012x over previous
import functools

import jax
import jax.numpy as jnp
from jax import lax
from jax.experimental import pallas as pl
from jax.experimental.pallas import tpu as pltpu
from jax.experimental.pallas import tpu_sc as plsc

B, V, N = 32, 4, 8192
H = W = 800
WPR = W // 32
BROWS = V * WPR

PB = 8


def _pack_body(wlo_ref, whi_ref, img_ref, bits_ref):
    for v in range(V):
        m = img_ref[0, v]
        lo = lax.dot_general(
            wlo_ref[...], m, (((1,), (0,)), ((), ())),
            preferred_element_type=jnp.float32,
        )
        hi = lax.dot_general(
            whi_ref[...], m, (((1,), (0,)), ((), ())),
            preferred_element_type=jnp.float32,
        )
        words = lo.astype(jnp.int32) | (hi.astype(jnp.int32) << 16)
        bits_ref[0, v] = words


def _pack(img_seg):
    j = jnp.arange(H)
    c = j >> 5
    s = j & 31
    onehot = (jnp.arange(WPR)[:, None] == c[None, :]).astype(jnp.float32)
    wlo = onehot * jnp.where(s < 16, 2.0 ** s, 0.0)[None, :]
    whi = onehot * jnp.where(s >= 16, 2.0 ** (s - 16), 0.0)[None, :]
    calls = []
    for k in range(B // PB):
        calls.append(pl.pallas_call(
            _pack_body,
            grid=(PB,),
            in_specs=[
                pl.BlockSpec((WPR, H), lambda b: (0, 0)),
                pl.BlockSpec((WPR, H), lambda b: (0, 0)),
                pl.BlockSpec((1, V, H, W), lambda b, k=k: (k * PB + b, 0, 0, 0)),
            ],
            out_specs=pl.BlockSpec((1, V, WPR, H), lambda b: (b, 0, 0, 0)),
            out_shape=jax.ShapeDtypeStruct((PB, V, WPR, H), jnp.int32),
        )(wlo.astype(jnp.float32), whi.astype(jnp.float32), img_seg))
    return calls


BB = 8
GRID = B // BB
ROWS = BB * V


def _proj_body(e_ref, kbd_ref, pc_ref, idx_ref):
    ones = jnp.ones((1, N), jnp.float32)
    parts = []
    for bp in range(BB):
        parts.append(pc_ref[bp])
        parts.append(ones)
    hom = jnp.concatenate(parts, axis=0)
    e = e_ref[0]
    pt = lax.dot_general(
        e, hom, (((1,), (0,)), ((), ())),
        preferred_element_type=jnp.float32,
    )
    p2 = lax.dot_general(
        kbd_ref[...], pt, (((1,), (0,)), ((), ())),
        preferred_element_type=jnp.float32,
    )
    px = p2[0:ROWS]
    py = p2[ROWS:2 * ROWS]
    pz = p2[2 * ROWS:3 * ROWS]
    xs = px / pz
    ys = py / pz
    xi = jnp.clip(jnp.round(xs).astype(jnp.int32), 0, W - 1)
    yi = jnp.clip(jnp.round(ys).astype(jnp.int32), 0, H - 1)
    vv = lax.broadcasted_iota(jnp.int32, (ROWS, N), 0) & 3
    brow = vv * WPR + (yi >> 5)
    idx_ref[...] = (brow << 15) | (xi << 5) | (yi & 31)


def _project(point_cloud, K, ext_trans):
    ext_r = ext_trans.reshape(GRID, BB, V, 3, 4)
    eye = jnp.eye(BB, dtype=ext_trans.dtype)
    e_all = jnp.einsum("gbvij,bc->gibvcj", ext_r, eye)
    e_all = e_all.reshape(GRID, 3 * ROWS, 4 * BB)
    kbd = jnp.kron(K, jnp.eye(ROWS, dtype=K.dtype))
    return pl.pallas_call(
        _proj_body,
        grid=(GRID,),
        in_specs=[
            pl.BlockSpec((1, 3 * ROWS, 4 * BB), lambda g: (g, 0, 0)),
            pl.BlockSpec((3 * ROWS, 3 * ROWS), lambda g: (0, 0)),
            pl.BlockSpec((BB, 3, N), lambda g: (g, 0, 0)),
        ],
        out_specs=pl.BlockSpec((ROWS, N), lambda g: (g, 0)),
        out_shape=jax.ShapeDtypeStruct((B * V, N), jnp.int32),
    )(e_all, kbd, point_cloud)


NC = 2
NW = 32


def _make_sc_body(k):
    def _sc_gather_body(bits_hbm, idx_hbm, out_hbm, slab_v, pv, ov, sem):
        wid = lax.axis_index("s") * NC + lax.axis_index("c")
        bl = wid >> 2
        v = wid & 3
        pltpu.sync_copy(bits_hbm.at[bl, v], slab_v)
        row = k * PB * V + wid
        pltpu.sync_copy(idx_hbm.at[row], pv)
        voff = v * WPR

        def body(i, c):
            o = i * 16
            p = pv[pl.ds(o, 16)]
            w = plsc.load_gather(slab_v, [(p >> 15) - voff, (p >> 5) & 1023])
            bit = (w >> (p & 31)) & 1
            ov[pl.ds(o, 16)] = bit.astype(jnp.float32)
            return c

        lax.fori_loop(0, N // 16, body, 0)
        pltpu.sync_copy(ov, out_hbm.at[wid])

    return _sc_gather_body


@functools.cache
def _sc_gather(k):
    mesh = plsc.VectorSubcoreMesh(core_axis_name="c", subcore_axis_name="s")
    return pl.kernel(
        _make_sc_body(k),
        out_type=jax.ShapeDtypeStruct((PB * V, N), jnp.float32),
        mesh=mesh,
        compiler_params=pltpu.CompilerParams(needs_layout_passes=False),
        scratch_types=[
            pltpu.VMEM((WPR, H), jnp.int32),
            pltpu.VMEM((N,), jnp.int32),
            pltpu.VMEM((N,), jnp.float32),
            pltpu.SemaphoreType.DMA,
        ],
    )


def kernel(point_cloud, img_seg, K, ext_trans):
    idx = _project(point_cloud, K, ext_trans)
    outs = []
    for k, bits_k in enumerate(_pack(img_seg)):
        outs.append(_sc_gather(k)(bits_k, idx))
    vals = jnp.concatenate(outs, axis=0)
    return vals.reshape(B, V, N).astype(bool)
